# baseline (speedup 1.0000x reference)
import jax
import jax.numpy as jnp
from jax.experimental import pallas as pl
from jax.experimental.pallas import tpu as pltpu

_VOCAB = 8192
_EMBED = 256
_BV = 512


def _body(p_ref, t_ref, o_ref, pn_ref, m_ref, a_ref):
    v = pl.program_id(0)

    @pl.when(v == 0)
    def _():
        p = p_ref[...]
        pnorm = jnp.sqrt(jnp.sum(p * p, axis=1, keepdims=True))
        pn_ref[...] = p / jnp.maximum(pnorm, 1e-12)

    t = t_ref[...]
    tnorm = jnp.sqrt(jnp.sum(t * t, axis=1, keepdims=True))
    tn = t / jnp.maximum(tnorm, 1e-12)
    sim = jax.lax.dot_general(
        pn_ref[...].astype(jnp.bfloat16), tn.astype(jnp.bfloat16),
        (((1,), (1,)), ((), ())),
        preferred_element_type=jnp.float32)
    m_c = jnp.max(sim, axis=1, keepdims=True)
    a_c = (jnp.argmax(sim, axis=1).astype(jnp.int32) + v * _BV).reshape(-1, 1)

    @pl.when(v == 0)
    def _():
        m_ref[...] = m_c
        a_ref[...] = a_c

    @pl.when(v == _VOCAB // (2 * _BV))
    def _():
        m_ref[...] = m_ref[...].astype(jnp.bfloat16).astype(jnp.float32)

    @pl.when(v > 0)
    def _():
        take = m_c > m_ref[...]
        m_ref[...] = jnp.where(take, m_c, m_ref[...])
        a_ref[...] = jnp.where(take, a_c, a_ref[...])

    @pl.when(v == pl.num_programs(0) - 1)
    def _():
        o_ref[...] = a_ref[...]


def kernel(projections, table, top_k=1):
    b, s, e = projections.shape
    rows = b * s
    p2 = projections.reshape(rows, e)
    out = pl.pallas_call(
        _body,
        grid=(_VOCAB // _BV,),
        in_specs=[
            pl.BlockSpec((rows, e), lambda v: (0, 0)),
            pl.BlockSpec((_BV, e), lambda v: (v, 0)),
        ],
        out_specs=pl.BlockSpec((rows, 1), lambda v: (0, 0)),
        out_shape=jax.ShapeDtypeStruct((rows, 1), jnp.int32),
        scratch_shapes=[
            pltpu.VMEM((rows, e), jnp.float32),
            pltpu.VMEM((rows, 1), jnp.float32),
            pltpu.VMEM((rows, 1), jnp.int32),
        ],
    )(p2, table)
    return out.reshape(b, s)


# speedup vs baseline: 2.7647x; 2.7647x over previous
import jax
import jax.numpy as jnp
from jax.experimental import pallas as pl
from jax.experimental.pallas import tpu as pltpu

_VOCAB = 8192
_EMBED = 256
_BV = 512
_NBLK = _VOCAB // _BV
_HALF_STEP = _NBLK // 2


def _body(p_ref, t_ref, o_ref, pn_ref, sim_ref, m_ref, a_ref):
    v = pl.program_id(0)
    nsteps = pl.num_programs(0)

    @pl.when(v == 0)
    def _():
        p = p_ref[...]
        pnorm = jnp.sqrt(jnp.sum(p * p, axis=1, keepdims=True))
        pn_ref[...] = (p / jnp.maximum(pnorm, 1e-12)).astype(jnp.bfloat16)

    @pl.when(v < nsteps - 1)
    def _():
        t = t_ref[...]
        tnorm = jnp.sqrt(jnp.sum(t * t, axis=1, keepdims=True))
        tn = (t / jnp.maximum(tnorm, 1e-12)).astype(jnp.bfloat16)
        sim_ref[v % 2] = jax.lax.dot_general(
            tn, pn_ref[...], (((1,), (1,)), ((), ())),
            preferred_element_type=jnp.float32)

    @pl.when(v > 0)
    def _():
        sim = sim_ref[(v - 1) % 2]
        m_c = jnp.max(sim, axis=0, keepdims=True)
        iota_s = jax.lax.broadcasted_iota(jnp.int32, (_BV, sim.shape[1]), 0)
        cand = jnp.where(sim == m_c, iota_s, _BV)
        a_c = jnp.min(cand, axis=0, keepdims=True) + (v - 1) * _BV

        @pl.when(v == 1)
        def _():
            m_ref[...] = m_c
            a_ref[...] = a_c

        @pl.when(v == _HALF_STEP + 1)
        def _():
            m_ref[...] = m_ref[...].astype(jnp.bfloat16).astype(jnp.float32)

        @pl.when(v > 1)
        def _():
            take = m_c > m_ref[...]
            m_ref[...] = jnp.where(take, m_c, m_ref[...])
            a_ref[...] = jnp.where(take, a_c, a_ref[...])

        @pl.when(v == nsteps - 1)
        def _():
            o_ref[...] = a_ref[...]


def kernel(projections, table, top_k=1):
    b, s, e = projections.shape
    rows = b * s
    p2 = projections.reshape(rows, e)
    out = pl.pallas_call(
        _body,
        grid=(_NBLK + 1,),
        in_specs=[
            pl.BlockSpec((rows, e), lambda v: (0, 0)),
            pl.BlockSpec((_BV, e), lambda v: (jnp.minimum(v, _NBLK - 1), 0)),
        ],
        out_specs=pl.BlockSpec((1, rows), lambda v: (0, 0)),
        out_shape=jax.ShapeDtypeStruct((1, rows), jnp.int32),
        scratch_shapes=[
            pltpu.VMEM((rows, _EMBED), jnp.bfloat16),
            pltpu.VMEM((2, _BV, rows), jnp.float32),
            pltpu.VMEM((1, rows), jnp.float32),
            pltpu.VMEM((1, rows), jnp.int32),
        ],
    )(p2, table)
    return out.reshape(b, s)


# speedup vs baseline: 2.9372x; 1.0624x over previous
import jax
import jax.numpy as jnp
from jax.experimental import pallas as pl
from jax.experimental.pallas import tpu as pltpu

_VOCAB = 8192
_EMBED = 256
_BV = 1024
_NBLK = _VOCAB // _BV
_HALF_STEP = _NBLK // 2


def _body(p_ref, t_ref, o_ref, pn_ref, sim_ref, m_ref, a_ref):
    v = pl.program_id(0)
    nsteps = pl.num_programs(0)

    @pl.when(v == 0)
    def _():
        p = p_ref[...]
        pnorm = jnp.sqrt(jnp.sum(p * p, axis=1, keepdims=True))
        pn_ref[...] = (p / jnp.maximum(pnorm, 1e-12)).astype(jnp.bfloat16)

    @pl.when(v < nsteps - 1)
    def _():
        t = t_ref[...]
        tnorm = jnp.sqrt(jnp.sum(t * t, axis=1, keepdims=True))
        tn = (t / jnp.maximum(tnorm, 1e-12)).astype(jnp.bfloat16)
        sim_ref[v % 2] = jax.lax.dot_general(
            tn, pn_ref[...], (((1,), (1,)), ((), ())),
            preferred_element_type=jnp.float32)

    @pl.when(v > 0)
    def _():
        sim = sim_ref[(v - 1) % 2]
        m_c = jnp.max(sim, axis=0, keepdims=True)
        iota_s = jax.lax.broadcasted_iota(jnp.int32, (_BV, sim.shape[1]), 0)
        cand = jnp.where(sim == m_c, iota_s, _BV)
        a_c = jnp.min(cand, axis=0, keepdims=True) + (v - 1) * _BV

        @pl.when(v == 1)
        def _():
            m_ref[...] = m_c
            a_ref[...] = a_c

        @pl.when(v == _HALF_STEP + 1)
        def _():
            m_ref[...] = m_ref[...].astype(jnp.bfloat16).astype(jnp.float32)

        @pl.when(v > 1)
        def _():
            take = m_c > m_ref[...]
            m_ref[...] = jnp.where(take, m_c, m_ref[...])
            a_ref[...] = jnp.where(take, a_c, a_ref[...])

        @pl.when(v == nsteps - 1)
        def _():
            o_ref[...] = a_ref[...]


def kernel(projections, table, top_k=1):
    b, s, e = projections.shape
    rows = b * s
    p2 = projections.reshape(rows, e)
    out = pl.pallas_call(
        _body,
        grid=(_NBLK + 1,),
        in_specs=[
            pl.BlockSpec((rows, e), lambda v: (0, 0)),
            pl.BlockSpec((_BV, e), lambda v: (jnp.minimum(v, _NBLK - 1), 0)),
        ],
        out_specs=pl.BlockSpec((1, rows), lambda v: (0, 0)),
        out_shape=jax.ShapeDtypeStruct((1, rows), jnp.int32),
        scratch_shapes=[
            pltpu.VMEM((rows, _EMBED), jnp.bfloat16),
            pltpu.VMEM((2, _BV, rows), jnp.float32),
            pltpu.VMEM((1, rows), jnp.float32),
            pltpu.VMEM((1, rows), jnp.int32),
        ],
    )(p2, table)
    return out.reshape(b, s)
